# baseline (speedup 1.0000x reference)
import functools

import jax
import jax.numpy as jnp
from jax import lax
from jax.experimental import pallas as pl
from jax.experimental.pallas import tpu as pltpu
from jax.experimental.pallas import tpu_sc as plsc

B, H, W, C = 8, 224, 224, 96
CP = 128
N = B * H * W
NC, NS, L = 2, 16, 16
NW = NC * NS
PW = N // NW
CH = 128
NCHUNK = PW // CH
HW = H * W


def _warp_body(table, fxh, fyh, out, fxv, fyv,
               i00, i01, i10, i11, w00, w01, w10, w11,
               t00, t01, t10, t11, outv, gsem):
    wid = lax.axis_index("s") * NC + lax.axis_index("c")
    wbase = wid * PW
    img_base = (wid >> 2) * HW

    def chunk(k, _):
        base = wbase + k * CH
        pltpu.sync_copy(fxh.at[pl.ds(base, CH)], fxv)
        pltpu.sync_copy(fyh.at[pl.ds(base, CH)], fyv)

        for g in range(CH // L):
            sl = pl.ds(g * L, L)
            x = fxv[sl]
            y = fyv[sl]
            xt = x.astype(jnp.int32)
            x0 = jnp.where(x < xt.astype(jnp.float32), xt - 1, xt)
            yt = y.astype(jnp.int32)
            y0 = jnp.where(y < yt.astype(jnp.float32), yt - 1, yt)
            fx = x - x0.astype(jnp.float32)
            fy = y - y0.astype(jnp.float32)
            inb = ((x0 >= 0) & (x0 <= W - 2)
                   & (y0 >= 0) & (y0 <= H - 2))
            m = jnp.where(inb, 1.0, 0.0).astype(jnp.float32)
            xb = jnp.clip(x0, 0, W - 2)
            yb = jnp.clip(y0, 0, H - 2)
            bidx = img_base + yb * W + xb
            i00[sl] = bidx
            i01[sl] = bidx + 1
            i10[sl] = bidx + W
            i11[sl] = bidx + W + 1
            gx1 = fx * m
            gx0 = m - gx1
            w00[sl] = gx0 * (1.0 - fy)
            w01[sl] = gx1 * (1.0 - fy)
            w10[sl] = gx0 * fy
            w11[sl] = gx1 * fy

        d0 = pltpu.async_copy(table.at[i00], t00, gsem)
        d1 = pltpu.async_copy(table.at[i01], t01, gsem)
        d2 = pltpu.async_copy(table.at[i10], t10, gsem)
        d3 = pltpu.async_copy(table.at[i11], t11, gsem)
        d0.wait()
        d1.wait()
        d2.wait()
        d3.wait()

        def splat(vec, lane):
            return jax.lax.gather(
                vec, lane[:, None],
                dimension_numbers=jax.lax.GatherDimensionNumbers(
                    offset_dims=(), collapsed_slice_dims=(0,),
                    start_index_map=(0,)),
                slice_sizes=(1,),
                mode=jax.lax.GatherScatterMode.PROMISE_IN_BOUNDS)

        def group(g, _):
            gs = pl.ds(g * L, L)
            v00 = w00[gs]
            v01 = w01[gs]
            v10 = w10[gs]
            v11 = w11[gs]
            for i in range(L):
                lane = jnp.full((L,), i, dtype=jnp.int32)
                s00 = splat(v00, lane)
                s01 = splat(v01, lane)
                s10 = splat(v10, lane)
                s11 = splat(v11, lane)
                p = g * L + i
                for cg in range(C // L):
                    cs = pl.ds(cg * L, L)
                    outv[p, cs] = (s00 * t00[p, cs] + s01 * t01[p, cs]
                                   + s10 * t10[p, cs] + s11 * t11[p, cs])
            return _

        lax.fori_loop(0, CH // L, group, None)
        pltpu.sync_copy(outv, out.at[pl.ds(base, CH)])
        return _

    lax.fori_loop(0, NCHUNK, chunk, None)


RB = 2048


def _pad_body(x_ref, o_ref):
    o_ref[...] = jnp.concatenate(
        [x_ref[...], jnp.zeros((RB, CP - C), jnp.float32)], axis=1)


@jax.jit
def kernel(img, flo):
    table = pl.pallas_call(
        _pad_body,
        grid=(N // RB,),
        in_specs=[pl.BlockSpec((RB, C), lambda i: (i, 0))],
        out_specs=pl.BlockSpec((RB, CP), lambda i: (i, 0)),
        out_shape=jax.ShapeDtypeStruct((N, CP), jnp.float32),
    )(img.reshape(N, C))
    xg = jnp.arange(W, dtype=jnp.float32)
    yg = jnp.arange(H, dtype=jnp.float32)
    fxh = (flo[..., 0] + xg[None, None, :]).reshape(N)
    fyh = (flo[..., 1] + yg[None, :, None]).reshape(N)

    mesh = plsc.VectorSubcoreMesh(core_axis_name="c", subcore_axis_name="s",
                                  num_cores=NC, num_subcores=NS)
    warp = pl.kernel(
        _warp_body,
        out_type=jax.ShapeDtypeStruct((N, C), jnp.float32),
        mesh=mesh,
        scratch_types=[
            pltpu.VMEM((CH,), jnp.float32),
            pltpu.VMEM((CH,), jnp.float32),
            pltpu.VMEM((CH,), jnp.int32),
            pltpu.VMEM((CH,), jnp.int32),
            pltpu.VMEM((CH,), jnp.int32),
            pltpu.VMEM((CH,), jnp.int32),
            pltpu.VMEM((CH,), jnp.float32),
            pltpu.VMEM((CH,), jnp.float32),
            pltpu.VMEM((CH,), jnp.float32),
            pltpu.VMEM((CH,), jnp.float32),
            pltpu.VMEM((CH, CP), jnp.float32),
            pltpu.VMEM((CH, CP), jnp.float32),
            pltpu.VMEM((CH, CP), jnp.float32),
            pltpu.VMEM((CH, CP), jnp.float32),
            pltpu.VMEM((CH, C), jnp.float32),
            pltpu.SemaphoreType.DMA,
        ],
    )
    out = warp(table, fxh, fyh)
    return out.reshape(B, H, W, C)


# speedup vs baseline: 1.1024x; 1.1024x over previous
import functools

import jax
import jax.numpy as jnp
from jax import lax
from jax.experimental import pallas as pl
from jax.experimental.pallas import tpu as pltpu
from jax.experimental.pallas import tpu_sc as plsc

B, H, W, C = 8, 224, 224, 96
CP = 128
N = B * H * W
NC, NS, L = 2, 16, 16
NW = NC * NS
PW = N // NW
CH = 128
NCHUNK = PW // CH
HW = H * W


def _warp_body(table, fxh, fyh, out, fxv, fyv,
               i00, i01, i10, i11, w00, w01, w10, w11,
               t00, t01, t10, t11, outv, gsem):
    wid = lax.axis_index("s") * NC + lax.axis_index("c")
    wbase = wid * PW
    img_base = (wid >> 2) * HW

    def chunk(k, _):
        base = wbase + k * CH
        pltpu.sync_copy(fxh.at[pl.ds(base, CH)], fxv)
        pltpu.sync_copy(fyh.at[pl.ds(base, CH)], fyv)

        for g in range(CH // L):
            sl = pl.ds(g * L, L)
            x = fxv[sl]
            y = fyv[sl]
            xt = x.astype(jnp.int32)
            x0 = jnp.where(x < xt.astype(jnp.float32), xt - 1, xt)
            yt = y.astype(jnp.int32)
            y0 = jnp.where(y < yt.astype(jnp.float32), yt - 1, yt)
            fx = x - x0.astype(jnp.float32)
            fy = y - y0.astype(jnp.float32)
            inb = ((x0 >= 0) & (x0 <= W - 2)
                   & (y0 >= 0) & (y0 <= H - 2))
            m = jnp.where(inb, 1.0, 0.0).astype(jnp.float32)
            xb = jnp.clip(x0, 0, W - 2)
            yb = jnp.clip(y0, 0, H - 2)
            bidx = img_base + yb * W + xb
            i00[sl] = bidx
            i01[sl] = bidx + 1
            i10[sl] = bidx + W
            i11[sl] = bidx + W + 1
            gx1 = fx * m
            gx0 = m - gx1
            w00[sl] = gx0 * (1.0 - fy)
            w01[sl] = gx1 * (1.0 - fy)
            w10[sl] = gx0 * fy
            w11[sl] = gx1 * fy

        d0 = pltpu.async_copy(table.at[i00], t00, gsem)
        d1 = pltpu.async_copy(table.at[i01], t01, gsem)
        d2 = pltpu.async_copy(table.at[i10], t10, gsem)
        d3 = pltpu.async_copy(table.at[i11], t11, gsem)
        d0.wait()
        d1.wait()
        d2.wait()
        d3.wait()

        def pixel(p, _):
            s00 = w00[pl.ds(p, 1)][0]
            s01 = w01[pl.ds(p, 1)][0]
            s10 = w10[pl.ds(p, 1)][0]
            s11 = w11[pl.ds(p, 1)][0]
            for cg in range(C // L):
                cs = pl.ds(cg * L, L)
                outv[p, cs] = (s00 * t00[p, cs] + s01 * t01[p, cs]
                               + s10 * t10[p, cs] + s11 * t11[p, cs])
            return _

        lax.fori_loop(0, CH, pixel, None)
        pltpu.sync_copy(outv, out.at[pl.ds(base, CH)])
        return _

    lax.fori_loop(0, NCHUNK, chunk, None)


@jax.jit
def kernel(img, flo):
    table = jnp.pad(img.reshape(N, C), ((0, 0), (0, CP - C)))
    xg = jnp.arange(W, dtype=jnp.float32)
    yg = jnp.arange(H, dtype=jnp.float32)
    fxh = (flo[..., 0] + xg[None, None, :]).reshape(N)
    fyh = (flo[..., 1] + yg[None, :, None]).reshape(N)

    mesh = plsc.VectorSubcoreMesh(core_axis_name="c", subcore_axis_name="s",
                                  num_cores=NC, num_subcores=NS)
    warp = pl.kernel(
        _warp_body,
        out_type=jax.ShapeDtypeStruct((N, C), jnp.float32),
        mesh=mesh,
        scratch_types=[
            pltpu.VMEM((CH,), jnp.float32),
            pltpu.VMEM((CH,), jnp.float32),
            pltpu.VMEM((CH,), jnp.int32),
            pltpu.VMEM((CH,), jnp.int32),
            pltpu.VMEM((CH,), jnp.int32),
            pltpu.VMEM((CH,), jnp.int32),
            pltpu.VMEM((CH,), jnp.float32),
            pltpu.VMEM((CH,), jnp.float32),
            pltpu.VMEM((CH,), jnp.float32),
            pltpu.VMEM((CH,), jnp.float32),
            pltpu.VMEM((CH, CP), jnp.float32),
            pltpu.VMEM((CH, CP), jnp.float32),
            pltpu.VMEM((CH, CP), jnp.float32),
            pltpu.VMEM((CH, CP), jnp.float32),
            pltpu.VMEM((CH, C), jnp.float32),
            pltpu.SemaphoreType.DMA,
        ],
    )
    out = warp(table, fxh, fyh)
    return out.reshape(B, H, W, C)
